# speedup vs baseline: 13.3421x; 13.3421x over previous
import functools

import jax
import jax.numpy as jnp
from jax.experimental import pallas as pl
from jax.experimental.pallas import tpu as pltpu

NUM_SEG = 64
D = 512
N = 50000
R = 1000
GRID = N // R


def _body(batch_ref, x_ref, out_ref, acc_ref):
    i = pl.program_id(0)

    @pl.when(i == 0)
    def _init():
        acc_ref[...] = jnp.zeros_like(acc_ref)

    b = batch_ref[0, 0, :]
    onehot = (b[:, None] == jax.lax.broadcasted_iota(jnp.int32, (R, NUM_SEG), 1)
              ).astype(jnp.float32)
    xa = jnp.concatenate(
        [x_ref[...], jnp.ones((R, 128), jnp.float32)], axis=1)
    acc_ref[...] += jax.lax.dot_general(
        onehot, xa, (((0,), (0,)), ((), ())),
        preferred_element_type=jnp.float32)

    @pl.when(i == GRID - 1)
    def _fin():
        cnt = jnp.clip(acc_ref[:, D:D + 1], 1.0, None)
        out_ref[...] = acc_ref[:, :D] / cnt


@jax.jit
def kernel(x, batch):
    batch3 = batch.astype(jnp.int32).reshape(GRID, 1, R)
    return pl.pallas_call(
        _body,
        grid=(GRID,),
        in_specs=[
            pl.BlockSpec((1, 1, R), lambda i: (i, 0, 0)),
            pl.BlockSpec((R, D), lambda i: (i, 0)),
        ],
        out_specs=pl.BlockSpec((NUM_SEG, D), lambda i: (0, 0)),
        out_shape=jax.ShapeDtypeStruct((NUM_SEG, D), jnp.float32),
        scratch_shapes=[pltpu.VMEM((NUM_SEG, D + 128), jnp.float32)],
        compiler_params=pltpu.CompilerParams(
            dimension_semantics=("arbitrary",)),
    )(batch3, x)
